# baseline (speedup 1.0000x reference)
import functools

import jax
import jax.numpy as jnp
from jax import lax
from jax.experimental import pallas as pl
from jax.experimental.pallas import tpu as pltpu
from jax.experimental.pallas import tpu_sc as plsc

VOCAB = 1000000
DIM = 64
NC = 2
NS = 16
NW = NC * NS

CHUNK = 128
B_TOK = 16384
L_SEQ = 50
NCHUNK_TOTAL = (B_TOK // CHUNK) * L_SEQ
NCH = NCHUNK_TOTAL // NW
BB_N = B_TOK // CHUNK


def _build_gather():
    mesh = plsc.VectorSubcoreMesh(core_axis_name="c", subcore_axis_name="s")

    @functools.partial(
        pl.kernel,
        mesh=mesh,
        out_type=jax.ShapeDtypeStruct((L_SEQ, DIM // 8, BB_N, 8, CHUNK),
                                      jnp.float32),
        compiler_params=pltpu.CompilerParams(use_tc_tiling_on_sc=False,
                                             needs_layout_passes=False),
        scratch_types=[
            pltpu.VMEM((NCH, CHUNK), jnp.int32),
            pltpu.VMEM((2, CHUNK, DIM), jnp.float32),
            pltpu.VMEM((2, DIM, CHUNK), jnp.float32),
            pltpu.SemaphoreType.DMA((2,)),
            pltpu.SemaphoreType.DMA((2,)),
        ],
    )
    def gather_kernel(idx_hbm, table_hbm, out_hbm, idx_v, bufs, tbufs,
                      gsem, wsem):
        c = lax.axis_index("c")
        s = lax.axis_index("s")
        wid = s * NC + c
        pltpu.sync_copy(idx_hbm.at[wid], idx_v)

        lane = lax.iota(jnp.int32, 16)
        rows = [lane + 16 * v for v in range(CHUNK // 16)]

        def fire_gather(j, slot):
            pltpu.async_copy(table_hbm.at[idx_v.at[j]], bufs.at[slot],
                             gsem.at[slot])

        def drain_gather(slot):
            pltpu.make_async_copy(table_hbm.at[pl.ds(0, CHUNK)],
                                  bufs.at[slot], gsem.at[slot]).wait()

        def fire_wb(j, slot):
            q = wid * NCH + j
            l = q // BB_N
            bb = lax.rem(q, BB_N)
            for tr in range(DIM // 8):
                pltpu.async_copy(tbufs.at[slot, pl.ds(8 * tr, 8)],
                                 out_hbm.at[l, tr, bb], wsem.at[slot])

        def drain_wb(slot):
            for tr in range(DIM // 8):
                pltpu.make_async_copy(tbufs.at[slot, pl.ds(0, 8)],
                                      out_hbm.at[0, 0, 0],
                                      wsem.at[slot]).wait()

        def transpose(slot):
            buf = bufs.at[slot]
            tbuf = tbufs.at[slot]

            @pl.loop(0, DIM)
            def _(d):
                col = jnp.full((16,), d, dtype=jnp.int32)
                for v in range(CHUNK // 16):
                    vals = plsc.load_gather(buf, [rows[v], col])
                    tbuf[d, pl.ds(16 * v, 16)] = vals

        fire_gather(0, 0)

        @pl.loop(0, NCH)
        def _(j):
            cur = lax.rem(j, 2)
            nxt = 1 - cur

            @pl.when(j + 1 < NCH)
            def _():
                fire_gather(j + 1, nxt)

            drain_gather(cur)

            @pl.when(j >= 2)
            def _():
                drain_wb(cur)

            transpose(cur)
            fire_wb(j, cur)

        drain_wb(0)
        drain_wb(1)

    return gather_kernel


_GATHER = _build_gather()


def kernel(x, embedding):
    idx = jnp.transpose(x).reshape(NW, NCH, CHUNK).astype(jnp.int32)
    out5 = _GATHER(idx, embedding)
    return out5.transpose(2, 4, 0, 1, 3).reshape(B_TOK, L_SEQ, DIM)
